# Initial kernel scaffold; baseline (speedup 1.0000x reference)
import jax
import jax.numpy as jnp
from jax.experimental import pallas as pl


def kernel(feat_data, adjs, Wf, bf, a_src, a_dest, W0, b0, W1, b1):
    raise NotImplementedError("write your pallas kernel here")


# speedup vs baseline: 1.3008x; 1.3008x over previous
import functools

import jax
import jax.numpy as jnp
from jax.experimental import pallas as pl

N = 4096
NFEAT = 128
NHID = 128
BLK = 256


def _prep_body(feat_ref, wft_ref, bf_ref, a2_ref, w0t_ref, b0_ref,
               f_ref, x0_ref):
    feat = feat_ref[...]
    h = jnp.dot(feat, wft_ref[...], preferred_element_type=jnp.float32)
    h = h + bf_ref[...]
    f_ref[...] = jnp.dot(h, a2_ref[...], preferred_element_type=jnp.float32)
    x0 = jnp.dot(feat, w0t_ref[...], preferred_element_type=jnp.float32)
    x0_ref[...] = x0 + b0_ref[...]


def _attn_factors(f1, f2, c1, c2):
    t = f1 + f2
    r = jnp.exp(0.8 * (f1 - c1))
    b1v = jnp.exp(f2 - c2)
    b2v = jnp.exp(0.2 * (f2 - c2) - 0.8 * (c1 + c2))
    e = jnp.where(t >= 0.0, r * b1v, jnp.broadcast_to(b2v, t.shape))
    s = jnp.sum(e, axis=1, keepdims=True)
    return e, s


def _attn1_body(f1_ref, f2_ref, c_ref, x_ref, o_ref):
    c1 = c_ref[0, 0]
    c2 = c_ref[0, 1]
    e, s = _attn_factors(f1_ref[...], f2_ref[...], c1, c2)
    p = jnp.dot(e, x_ref[...], preferred_element_type=jnp.float32)
    y = p / s
    o_ref[...] = jnp.where(y > 0.0, y, jnp.exp(y) - 1.0)


def _attn2_body(f1_ref, f2_ref, c_ref, x_ref, w1t_ref, b1_ref, o_ref):
    c1 = c_ref[0, 0]
    c2 = c_ref[0, 1]
    e, s = _attn_factors(f1_ref[...], f2_ref[...], c1, c2)
    p = jnp.dot(e, x_ref[...], preferred_element_type=jnp.float32)
    y = p / s
    z = jnp.dot(y, w1t_ref[...], preferred_element_type=jnp.float32)
    z = z + b1_ref[...]
    o_ref[...] = jnp.where(z > 0.0, z, jnp.exp(z) - 1.0)


@jax.jit
def kernel(feat_data, adjs, Wf, bf, a_src, a_dest, W0, b0, W1, b1):
    del adjs
    nblk = N // BLK
    a2 = jnp.concatenate([a_src, a_dest], axis=1)

    f, x0 = pl.pallas_call(
        _prep_body,
        grid=(nblk,),
        in_specs=[
            pl.BlockSpec((BLK, NFEAT), lambda i: (i, 0)),
            pl.BlockSpec((NFEAT, NHID), lambda i: (0, 0)),
            pl.BlockSpec((1, NHID), lambda i: (0, 0)),
            pl.BlockSpec((NHID, 2), lambda i: (0, 0)),
            pl.BlockSpec((NFEAT, NHID), lambda i: (0, 0)),
            pl.BlockSpec((1, NHID), lambda i: (0, 0)),
        ],
        out_specs=[
            pl.BlockSpec((BLK, 2), lambda i: (i, 0)),
            pl.BlockSpec((BLK, NHID), lambda i: (i, 0)),
        ],
        out_shape=[
            jax.ShapeDtypeStruct((N, 2), jnp.float32),
            jax.ShapeDtypeStruct((N, NHID), jnp.float32),
        ],
    )(feat_data, Wf.T, bf.reshape(1, NHID), a2, W0.T, b0.reshape(1, NHID))

    c = jnp.max(f, axis=0).reshape(1, 2)
    f1c = f[:, 0:1]
    f2r = f[:, 1].reshape(1, N)

    attn_specs = [
        pl.BlockSpec((BLK, 1), lambda i: (i, 0)),
        pl.BlockSpec((1, N), lambda i: (0, 0)),
        pl.BlockSpec((1, 2), lambda i: (0, 0)),
        pl.BlockSpec((N, NHID), lambda i: (0, 0)),
    ]

    y0 = pl.pallas_call(
        _attn1_body,
        grid=(nblk,),
        in_specs=attn_specs,
        out_specs=pl.BlockSpec((BLK, NHID), lambda i: (i, 0)),
        out_shape=jax.ShapeDtypeStruct((N, NHID), jnp.float32),
    )(f1c, f2r, c, x0)

    out = pl.pallas_call(
        _attn2_body,
        grid=(nblk,),
        in_specs=attn_specs + [
            pl.BlockSpec((NHID, NHID), lambda i: (0, 0)),
            pl.BlockSpec((1, NHID), lambda i: (0, 0)),
        ],
        out_specs=pl.BlockSpec((BLK, NHID), lambda i: (i, 0)),
        out_shape=jax.ShapeDtypeStruct((N, NHID), jnp.float32),
    )(f1c, f2r, c, y0, W1.T, b1.reshape(1, NHID))

    return out


# speedup vs baseline: 1.4225x; 1.0935x over previous
import functools

import jax
import jax.numpy as jnp
from jax.experimental import pallas as pl

N = 4096
NFEAT = 128
NHID = 128
BLK = 256


def _prep_body(feat_ref, wft_ref, bf_ref, a2_ref, w0t_ref, b0_ref,
               f_ref, x0_ref):
    feat = feat_ref[...]
    h = jnp.dot(feat, wft_ref[...], preferred_element_type=jnp.float32)
    h = h + bf_ref[...]
    f_ref[...] = jnp.dot(h, a2_ref[...], preferred_element_type=jnp.float32)
    x0 = jnp.dot(feat, w0t_ref[...], preferred_element_type=jnp.float32)
    x0_ref[...] = x0 + b0_ref[...]


def _attn_factors(f1, f2, c1, c2):
    r = jnp.exp(0.8 * (f1 - c1))
    b1v = jnp.exp(f2 - c2)
    b2v = jnp.exp(0.2 * (f2 - c2) - 0.8 * (c1 + c2))
    e = jnp.maximum(r * b1v, b2v)
    s = jnp.sum(e, axis=1, keepdims=True)
    return e, s


def _attn1_body(f1_ref, f2_ref, c_ref, x_ref, o_ref):
    c1 = c_ref[0, 0]
    c2 = c_ref[0, 1]
    e, s = _attn_factors(f1_ref[...], f2_ref[...], c1, c2)
    p = jnp.dot(e, x_ref[...], preferred_element_type=jnp.float32)
    y = p / s
    o_ref[...] = jnp.where(y > 0.0, y, jnp.exp(y) - 1.0)


def _attn2_body(f1_ref, f2_ref, c_ref, x_ref, w1t_ref, b1_ref, o_ref):
    c1 = c_ref[0, 0]
    c2 = c_ref[0, 1]
    e, s = _attn_factors(f1_ref[...], f2_ref[...], c1, c2)
    p = jnp.dot(e, x_ref[...], preferred_element_type=jnp.float32)
    y = p / s
    z = jnp.dot(y, w1t_ref[...], preferred_element_type=jnp.float32)
    z = z + b1_ref[...]
    o_ref[...] = jnp.where(z > 0.0, z, jnp.exp(z) - 1.0)


@jax.jit
def kernel(feat_data, adjs, Wf, bf, a_src, a_dest, W0, b0, W1, b1):
    del adjs
    nblk = N // BLK
    a2 = jnp.concatenate([a_src, a_dest], axis=1)

    f, x0 = pl.pallas_call(
        _prep_body,
        grid=(nblk,),
        in_specs=[
            pl.BlockSpec((BLK, NFEAT), lambda i: (i, 0)),
            pl.BlockSpec((NFEAT, NHID), lambda i: (0, 0)),
            pl.BlockSpec((1, NHID), lambda i: (0, 0)),
            pl.BlockSpec((NHID, 2), lambda i: (0, 0)),
            pl.BlockSpec((NFEAT, NHID), lambda i: (0, 0)),
            pl.BlockSpec((1, NHID), lambda i: (0, 0)),
        ],
        out_specs=[
            pl.BlockSpec((BLK, 2), lambda i: (i, 0)),
            pl.BlockSpec((BLK, NHID), lambda i: (i, 0)),
        ],
        out_shape=[
            jax.ShapeDtypeStruct((N, 2), jnp.float32),
            jax.ShapeDtypeStruct((N, NHID), jnp.float32),
        ],
    )(feat_data, Wf.T, bf.reshape(1, NHID), a2, W0.T, b0.reshape(1, NHID))

    c = jnp.max(f, axis=0).reshape(1, 2)
    f1c = f[:, 0:1]
    f2r = f[:, 1].reshape(1, N)

    attn_specs = [
        pl.BlockSpec((BLK, 1), lambda i: (i, 0)),
        pl.BlockSpec((1, N), lambda i: (0, 0)),
        pl.BlockSpec((1, 2), lambda i: (0, 0)),
        pl.BlockSpec((N, NHID), lambda i: (0, 0)),
    ]

    y0 = pl.pallas_call(
        _attn1_body,
        grid=(nblk,),
        in_specs=attn_specs,
        out_specs=pl.BlockSpec((BLK, NHID), lambda i: (i, 0)),
        out_shape=jax.ShapeDtypeStruct((N, NHID), jnp.float32),
    )(f1c, f2r, c, x0)

    out = pl.pallas_call(
        _attn2_body,
        grid=(nblk,),
        in_specs=attn_specs + [
            pl.BlockSpec((NHID, NHID), lambda i: (0, 0)),
            pl.BlockSpec((1, NHID), lambda i: (0, 0)),
        ],
        out_specs=pl.BlockSpec((BLK, NHID), lambda i: (i, 0)),
        out_shape=jax.ShapeDtypeStruct((N, NHID), jnp.float32),
    )(f1c, f2r, c, y0, W1.T, b1.reshape(1, NHID))

    return out
